# Initial kernel scaffold; baseline (speedup 1.0000x reference)
import jax
import jax.numpy as jnp
from jax.experimental import pallas as pl


def kernel(x, edge_index, params):
    raise NotImplementedError("write your pallas kernel here")


# speedup vs baseline: 9.3219x; 9.3219x over previous
import functools

import jax
import jax.numpy as jnp
from jax import lax
from jax.experimental import pallas as pl
from jax.experimental.pallas import tpu as pltpu
from jax.experimental.pallas import tpu_sc as plsc

N_NODES = 10080
N_EDGES = 322560
BATCH = 360
NPG = 28
SEQ = 128
BL = BATCH * SEQ

NC = 2
NS = 16
NW = NC * NS
EPT = N_EDGES // NW
K_EDGE = 120
NCHUNK = EPT // K_EDGE
NPT = N_NODES // NS
NPAD = 10240
LANES = 16
EPS = 1e-5

_SC_MESH = plsc.VectorSubcoreMesh(core_axis_name="c", subcore_axis_name="s",
                                  num_cores=NC, num_subcores=NS)


@functools.partial(
    pl.kernel,
    out_type=jax.ShapeDtypeStruct((NW, NPAD), jnp.float32),
    mesh=_SC_MESH,
    scratch_types=[
        pltpu.VMEM((EPT,), jnp.int32),
        pltpu.VMEM((NPAD,), jnp.float32),
    ],
    compiler_params=pltpu.CompilerParams(needs_layout_passes=False),
)
def _deg_counts(dst_hbm, out_hbm, idx_v, hist_v):
    cid = lax.axis_index("c")
    sid = lax.axis_index("s")
    wid = cid * NS + sid

    zero16 = jnp.zeros((LANES,), jnp.float32)

    def _zero(i, c):
        hist_v[pl.ds(i * LANES, LANES)] = zero16
        return c

    lax.fori_loop(0, NPAD // LANES, _zero, 0)

    pltpu.sync_copy(dst_hbm.at[pl.ds(wid * EPT, EPT)], idx_v)

    ones16 = jnp.ones((LANES,), jnp.float32)

    def _accum(i, c):
        idx = idx_v[pl.ds(i * LANES, LANES)]
        plsc.addupdate_scatter(hist_v, [idx], ones16)
        return c

    lax.fori_loop(0, EPT // LANES, _accum, 0)

    pltpu.sync_copy(hist_v, out_hbm.at[wid])


def _make_msg_kernel(d):
    @functools.partial(
        pl.kernel,
        out_type=jax.ShapeDtypeStruct((NC, N_NODES, d), jnp.float32),
        mesh=_SC_MESH,
        scratch_types=[
            pltpu.VMEM((K_EDGE,), jnp.int32),
            pltpu.VMEM((K_EDGE,), jnp.int32),
            pltpu.VMEM((K_EDGE, d), jnp.float32),
            pltpu.VMEM_SHARED((N_NODES, d), jnp.float32),
            pltpu.SemaphoreType.DMA,
        ],
        compiler_params=pltpu.CompilerParams(needs_layout_passes=False,
                                             use_tc_tiling_on_sc=False),
    )
    def _msg(src_hbm, dst_hbm, g_hbm, out_hbm, si_v, di_v, rows_v, acc_sh, sem):
        cid = lax.axis_index("c")
        sid = lax.axis_index("s")
        wid = cid * NS + sid
        row0 = sid * NPT

        pltpu.sync_copy(g_hbm.at[pl.ds(row0, NPT)], acc_sh.at[pl.ds(row0, NPT)])
        plsc.subcore_barrier()

        def _chunk(j, c):
            base = wid * EPT + j * K_EDGE
            pltpu.sync_copy(src_hbm.at[pl.ds(base, K_EDGE)], si_v)
            pltpu.async_copy(g_hbm.at[si_v], rows_v, sem).wait()
            pltpu.sync_copy(dst_hbm.at[pl.ds(base, K_EDGE)], di_v)
            pltpu.sync_copy(rows_v, acc_sh.at[di_v], add=True)
            return c

        lax.fori_loop(0, NCHUNK, _chunk, 0)
        plsc.subcore_barrier()

        pltpu.sync_copy(acc_sh.at[pl.ds(row0, NPT)],
                        out_hbm.at[cid, pl.ds(row0, NPT)])

    return _msg


_msg64 = _make_msg_kernel(64)
_msg128 = _make_msg_kernel(128)


def _t0_body(x_ref, w_ref, dis_ref, out_ref):
    out_ref[...] = jnp.dot(x_ref[...], w_ref[...],
                           preferred_element_type=jnp.float32) * dis_ref[...]


def _t0(x, w, dis):
    return pl.pallas_call(
        _t0_body,
        out_shape=jax.ShapeDtypeStruct((N_NODES, w.shape[1]), jnp.float32),
    )(x, w, dis)


def _gcn_post(p0, p1, g, dis, b, gam, bet):
    s = (p0 + p1 - g) * dis + b
    m = jnp.mean(s, axis=0, keepdims=True)
    c = s - m
    v = jnp.mean(c * c, axis=0, keepdims=True)
    return jnp.maximum(gam * c * lax.rsqrt(v + EPS) + bet, 0.0)


def _tmid_body(p0_ref, p1_ref, g_ref, dis_ref, b_ref, gam_ref, bet_ref,
               w_ref, out_ref):
    h = _gcn_post(p0_ref[...], p1_ref[...], g_ref[...], dis_ref[...],
                  b_ref[...], gam_ref[...], bet_ref[...])
    out_ref[...] = jnp.dot(h, w_ref[...],
                           preferred_element_type=jnp.float32) * dis_ref[...]


def _tmid(p0, p1, g, dis, b, gam, bet, w):
    return pl.pallas_call(
        _tmid_body,
        out_shape=jax.ShapeDtypeStruct((N_NODES, w.shape[1]), jnp.float32),
    )(p0, p1, g, dis, b, gam, bet, w)


def _tlast_body(p0_ref, p1_ref, g_ref, dis_ref, b_ref, gam_ref, bet_ref,
                out_ref):
    out_ref[...] = _gcn_post(p0_ref[...], p1_ref[...], g_ref[...], dis_ref[...],
                             b_ref[...], gam_ref[...], bet_ref[...])


def _tlast(p0, p1, g, dis, b, gam, bet):
    return pl.pallas_call(
        _tlast_body,
        out_shape=jax.ShapeDtypeStruct((N_NODES, 128), jnp.float32),
    )(p0, p1, g, dis, b, gam, bet)


CB = 5760
NCB = BL // CB


def _shift_cols(x, s):
    if s == 0:
        return x
    w = x.shape[1]
    z = jnp.zeros((x.shape[0], s), jnp.float32)
    xs = jnp.concatenate([z, x[:, :w - s]], axis=1)
    col = lax.broadcasted_iota(jnp.int32, (1, w), 1)
    return xs * (col % SEQ >= s).astype(jnp.float32)


def _conv_cf(x, wk, b, dil):
    o = b
    for k in range(3):
        o = o + jnp.dot(wk[k], _shift_cols(x, (2 - k) * dil),
                        preferred_element_type=jnp.float32)
    return o


def _bn_apply(x, s1, s2, gam, bet):
    m = s1 * (1.0 / BL)
    v = s2 * (1.0 / BL) - m * m
    return jnp.maximum(gam * (x - m) * lax.rsqrt(v + EPS) + bet, 0.0)


def _make_conv_stats(cin, cout, dil, pre_bn):
    def _body(*refs):
        if pre_bn:
            (x_ref, ps1_ref, ps2_ref, pg_ref, pb_ref,
             w0_ref, w1_ref, w2_ref, b_ref, o_ref, s1_ref, s2_ref) = refs
            x = _bn_apply(x_ref[...], ps1_ref[...], ps2_ref[...],
                          pg_ref[...], pb_ref[...])
        else:
            x_ref, w0_ref, w1_ref, w2_ref, b_ref, o_ref, s1_ref, s2_ref = refs
            x = x_ref[...]
        o = _conv_cf(x, [w0_ref[...], w1_ref[...], w2_ref[...]],
                     b_ref[...], dil)
        o_ref[...] = o

        @pl.when(pl.program_id(0) == 0)
        def _():
            s1_ref[...] = jnp.zeros_like(s1_ref)
            s2_ref[...] = jnp.zeros_like(s2_ref)

        s1_ref[...] += jnp.sum(o, axis=1, keepdims=True)
        s2_ref[...] += jnp.sum(o * o, axis=1, keepdims=True)

    col_spec = lambda c: pl.BlockSpec((c, CB), lambda j: (0, j))
    full = lambda a, b: pl.BlockSpec((a, b), lambda j: (0, 0))
    in_specs = [col_spec(cin)]
    if pre_bn:
        in_specs += [full(cin, 1)] * 4
    in_specs += [full(cout, cin)] * 3 + [full(cout, 1)]

    def _call(x, w3, b, pre=()):
        return pl.pallas_call(
            _body,
            grid=(NCB,),
            in_specs=in_specs,
            out_specs=[col_spec(cout), full(cout, 1), full(cout, 1)],
            out_shape=[jax.ShapeDtypeStruct((cout, BL), jnp.float32),
                       jax.ShapeDtypeStruct((cout, 1), jnp.float32),
                       jax.ShapeDtypeStruct((cout, 1), jnp.float32)],
        )(x, *pre, w3[:, :, 0], w3[:, :, 1], w3[:, :, 2], b[:, None])

    return _call


def _make_res_combine(cin, cout):
    def _body(o2_ref, s1_ref, s2_ref, g_ref, be_ref, x_ref, wd_ref, bd_ref,
              out_ref):
        a2 = _bn_apply(o2_ref[...], s1_ref[...], s2_ref[...],
                       g_ref[...], be_ref[...])
        res = jnp.dot(wd_ref[...], x_ref[...],
                      preferred_element_type=jnp.float32) + bd_ref[...]
        out_ref[...] = jnp.maximum(a2 + res, 0.0)

    col_spec = lambda c: pl.BlockSpec((c, CB), lambda j: (0, j))
    full = lambda a, b: pl.BlockSpec((a, b), lambda j: (0, 0))

    def _call(o2, s1, s2, g, be, x, wd, bd):
        return pl.pallas_call(
            _body,
            grid=(NCB,),
            in_specs=[col_spec(cout), full(cout, 1), full(cout, 1),
                      full(cout, 1), full(cout, 1), col_spec(cin),
                      full(cout, cin), full(cout, 1)],
            out_specs=col_spec(cout),
            out_shape=jax.ShapeDtypeStruct((cout, BL), jnp.float32),
        )(o2, s1, s2, g, be, x, wd, bd)

    return _call


def _make_tblock(cin, cout, dil):
    conv1 = _make_conv_stats(cin, cout, dil, pre_bn=False)
    conv2 = _make_conv_stats(cout, cout, dil, pre_bn=True)
    comb = _make_res_combine(cin, cout)

    def _call(x, p, pre):
        o1, a1, a2 = conv1(x, p[pre + '_w1'], p[pre + '_b1'])
        o2, c1, c2 = conv2(o1, p[pre + '_w2'], p[pre + '_b2'],
                           pre=(a1, a2, p[pre + '_bn1_g'][:, None],
                                p[pre + '_bn1_b'][:, None]))
        return comb(o2, c1, c2, p[pre + '_bn2_g'][:, None],
                    p[pre + '_bn2_b'][:, None], x,
                    p[pre + '_down_w'][:, :, 0], p[pre + '_down_b'][:, None])

    return _call


_tblock0 = _make_tblock(NPG, 128, 1)
_tblock1 = _make_tblock(128, 64, 2)
_tblock2 = _make_tblock(64, 32, 4)


def _head_body(t_ref, w1_ref, b1_ref, w2_ref, b2_ref, out_ref):
    wc = jnp.dot(w1_ref[...], w2_ref[...], preferred_element_type=jnp.float32)
    bc = jnp.dot(b1_ref[...], w2_ref[...],
                 preferred_element_type=jnp.float32) + b2_ref[...]
    out_ref[...] = jnp.dot(t_ref[...], wc,
                           preferred_element_type=jnp.float32) + bc


def _head(t, w1, b1, w2, b2):
    return pl.pallas_call(
        _head_body,
        out_shape=jax.ShapeDtypeStruct((BATCH, 1), jnp.float32),
    )(t, w1, b1, w2, b2)


def kernel(x, edge_index, params):
    p = params
    src = edge_index[0]
    dst = edge_index[1]

    deg_part = _deg_counts(dst)
    deg = jnp.sum(deg_part[:, :N_NODES], axis=0) + 1.0
    dis = lax.rsqrt(deg)[:, None]

    g1 = _t0(x, p['gcn1_w'], dis)
    pt = _msg64(src, dst, g1)
    g2 = _tmid(pt[0], pt[1], g1, dis, p['gcn1_b'][None, :],
               p['bn1_g'][None, :], p['bn1_b'][None, :], p['gcn2_w'])
    pt = _msg128(src, dst, g2)
    g3 = _tmid(pt[0], pt[1], g2, dis, p['gcn2_b'][None, :],
               p['bn2_g'][None, :], p['bn2_b'][None, :], p['gcn3_w'])
    pt = _msg128(src, dst, g3)
    h3 = _tlast(pt[0], pt[1], g3, dis, p['gcn3_b'][None, :],
                p['bn3_g'][None, :], p['bn3_b'][None, :])

    x0 = h3.reshape(BATCH, NPG, SEQ).transpose(1, 0, 2).reshape(NPG, BL)
    x1 = _tblock0(x0, p, 'tcn0')
    x2 = _tblock1(x1, p, 'tcn1')
    x3 = _tblock2(x2, p, 'tcn2')

    t = x3.reshape(32, BATCH, SEQ)[:, :, SEQ - 1].T
    return _head(t, p['fc1_w'], p['fc1_b'][None, :], p['fc_w'],
                 p['fc_b'][None, :])


# speedup vs baseline: 14.6342x; 1.5699x over previous
import functools

import jax
import jax.numpy as jnp
from jax import lax
from jax.experimental import pallas as pl
from jax.experimental.pallas import tpu as pltpu
from jax.experimental.pallas import tpu_sc as plsc

N_NODES = 10080
N_EDGES = 322560
BATCH = 360
NPG = 28
SEQ = 128
BL = BATCH * SEQ

NC = 2
NS = 16
NW = NC * NS
EPT = N_EDGES // NW
K_EDGE = 112
NCHUNK = EPT // K_EDGE
NPT = N_NODES // NS
NPAD = 10240
LANES = 16
EPS = 1e-5

_SC_MESH = plsc.VectorSubcoreMesh(core_axis_name="c", subcore_axis_name="s",
                                  num_cores=NC, num_subcores=NS)


@functools.partial(
    pl.kernel,
    out_type=jax.ShapeDtypeStruct((NW, NPAD), jnp.float32),
    mesh=_SC_MESH,
    scratch_types=[
        pltpu.VMEM((EPT,), jnp.int32),
        pltpu.VMEM((NPAD,), jnp.float32),
    ],
    compiler_params=pltpu.CompilerParams(needs_layout_passes=False),
)
def _deg_counts(dst_hbm, out_hbm, idx_v, hist_v):
    cid = lax.axis_index("c")
    sid = lax.axis_index("s")
    wid = cid * NS + sid

    zero16 = jnp.zeros((LANES,), jnp.float32)

    def _zero(i, c):
        hist_v[pl.ds(i * LANES, LANES)] = zero16
        return c

    lax.fori_loop(0, NPAD // LANES, _zero, 0)

    pltpu.sync_copy(dst_hbm.at[pl.ds(wid * EPT, EPT)], idx_v)

    ones16 = jnp.ones((LANES,), jnp.float32)

    def _accum(i, c):
        idx = idx_v[pl.ds(i * LANES, LANES)]
        plsc.addupdate_scatter(hist_v, [idx], ones16)
        return c

    lax.fori_loop(0, EPT // LANES, _accum, 0)

    pltpu.sync_copy(hist_v, out_hbm.at[wid])


def _make_msg_kernel(d):
    @functools.partial(
        pl.kernel,
        out_type=jax.ShapeDtypeStruct((NC, N_NODES, d), jnp.float32),
        mesh=_SC_MESH,
        scratch_types=[
            pltpu.VMEM((EPT,), jnp.int32),
            pltpu.VMEM((NCHUNK, K_EDGE), jnp.int32),
            pltpu.VMEM((K_EDGE, d), jnp.float32),
            pltpu.VMEM((K_EDGE, d), jnp.float32),
            pltpu.VMEM_SHARED((N_NODES, d), jnp.float32),
            pltpu.SemaphoreType.DMA,
            pltpu.SemaphoreType.DMA,
        ],
        compiler_params=pltpu.CompilerParams(needs_layout_passes=False,
                                             use_tc_tiling_on_sc=False),
    )
    def _msg(src_hbm, dst2_hbm, g_hbm, out_hbm, si_v, di_v, rows0_v, rows1_v,
             acc_sh, sem0, sem1):
        cid = lax.axis_index("c")
        sid = lax.axis_index("s")
        wid = cid * NS + sid
        row0 = sid * NPT

        pltpu.sync_copy(src_hbm.at[pl.ds(wid * EPT, EPT)], si_v)
        pltpu.sync_copy(dst2_hbm.at[pl.ds(wid * NCHUNK, NCHUNK)], di_v)
        pltpu.sync_copy(g_hbm.at[pl.ds(row0, NPT)], acc_sh.at[pl.ds(row0, NPT)])
        plsc.subcore_barrier()

        def _gather(j, rref, sem):
            return pltpu.make_async_copy(
                g_hbm.at[si_v.at[pl.ds(j * K_EDGE, K_EDGE)]], rref, sem)

        _gather(0, rows0_v, sem0).start()
        _gather(1, rows1_v, sem1).start()

        def _step(j, rref, sem):
            _gather(j, rref, sem).wait()
            pltpu.sync_copy(rref, acc_sh.at[di_v.at[j]], add=True)

            @pl.when(j + 2 < NCHUNK)
            def _():
                _gather(j + 2, rref, sem).start()

        def _chunk2(j2, c):
            _step(j2 * 2, rows0_v, sem0)
            _step(j2 * 2 + 1, rows1_v, sem1)
            return c

        lax.fori_loop(0, NCHUNK // 2, _chunk2, 0)
        plsc.subcore_barrier()

        pltpu.sync_copy(acc_sh.at[pl.ds(row0, NPT)],
                        out_hbm.at[cid, pl.ds(row0, NPT)])

    return _msg


_msg128 = _make_msg_kernel(128)


def _t0_body(x_ref, w_ref, dis_ref, out_ref):
    out_ref[...] = jnp.dot(x_ref[...], w_ref[...],
                           preferred_element_type=jnp.float32) * dis_ref[...]


def _t0(x, w, dis):
    return pl.pallas_call(
        _t0_body,
        out_shape=jax.ShapeDtypeStruct((N_NODES, w.shape[1]), jnp.float32),
    )(x, w, dis)


def _gcn_post(p0, p1, g, dis, b, gam, bet):
    s = (p0 + p1 - g) * dis + b
    m = jnp.mean(s, axis=0, keepdims=True)
    c = s - m
    v = jnp.mean(c * c, axis=0, keepdims=True)
    return jnp.maximum(gam * c * lax.rsqrt(v + EPS) + bet, 0.0)


def _tmid_body(p0_ref, p1_ref, g_ref, dis_ref, b_ref, gam_ref, bet_ref,
               w_ref, out_ref):
    h = _gcn_post(p0_ref[...], p1_ref[...], g_ref[...], dis_ref[...],
                  b_ref[...], gam_ref[...], bet_ref[...])
    out_ref[...] = jnp.dot(h, w_ref[...],
                           preferred_element_type=jnp.float32) * dis_ref[...]


def _tmid(p0, p1, g, dis, b, gam, bet, w):
    return pl.pallas_call(
        _tmid_body,
        out_shape=jax.ShapeDtypeStruct((N_NODES, w.shape[1]), jnp.float32),
    )(p0, p1, g, dis, b, gam, bet, w)


def _tlast_body(p0_ref, p1_ref, g_ref, dis_ref, b_ref, gam_ref, bet_ref,
                out_ref):
    out_ref[...] = _gcn_post(p0_ref[...], p1_ref[...], g_ref[...], dis_ref[...],
                             b_ref[...], gam_ref[...], bet_ref[...])


def _tlast(p0, p1, g, dis, b, gam, bet):
    return pl.pallas_call(
        _tlast_body,
        out_shape=jax.ShapeDtypeStruct((N_NODES, 128), jnp.float32),
    )(p0, p1, g, dis, b, gam, bet)


CB = 5760
NCB = BL // CB


def _shift_cols(x, s):
    if s == 0:
        return x
    w = x.shape[1]
    z = jnp.zeros((x.shape[0], s), jnp.float32)
    xs = jnp.concatenate([z, x[:, :w - s]], axis=1)
    col = lax.broadcasted_iota(jnp.int32, (1, w), 1)
    return xs * (col % SEQ >= s).astype(jnp.float32)


def _conv_cf(x, wk, b, dil):
    o = b
    for k in range(3):
        o = o + jnp.dot(wk[k], _shift_cols(x, (2 - k) * dil),
                        preferred_element_type=jnp.float32)
    return o


def _bn_apply(x, s1, s2, gam, bet):
    m = s1 * (1.0 / BL)
    v = s2 * (1.0 / BL) - m * m
    return jnp.maximum(gam * (x - m) * lax.rsqrt(v + EPS) + bet, 0.0)


def _make_conv_stats(cin, cout, dil, pre_bn):
    def _body(*refs):
        if pre_bn:
            (x_ref, ps1_ref, ps2_ref, pg_ref, pb_ref,
             w0_ref, w1_ref, w2_ref, b_ref, o_ref, s1_ref, s2_ref) = refs
            x = _bn_apply(x_ref[...], ps1_ref[...], ps2_ref[...],
                          pg_ref[...], pb_ref[...])
        else:
            x_ref, w0_ref, w1_ref, w2_ref, b_ref, o_ref, s1_ref, s2_ref = refs
            x = x_ref[...]
        o = _conv_cf(x, [w0_ref[...], w1_ref[...], w2_ref[...]],
                     b_ref[...], dil)
        o_ref[...] = o

        @pl.when(pl.program_id(0) == 0)
        def _():
            s1_ref[...] = jnp.zeros_like(s1_ref)
            s2_ref[...] = jnp.zeros_like(s2_ref)

        s1_ref[...] += jnp.sum(o, axis=1, keepdims=True)
        s2_ref[...] += jnp.sum(o * o, axis=1, keepdims=True)

    col_spec = lambda c: pl.BlockSpec((c, CB), lambda j: (0, j))
    full = lambda a, b: pl.BlockSpec((a, b), lambda j: (0, 0))
    in_specs = [col_spec(cin)]
    if pre_bn:
        in_specs += [full(cin, 1)] * 4
    in_specs += [full(cout, cin)] * 3 + [full(cout, 1)]

    def _call(x, w3, b, pre=()):
        return pl.pallas_call(
            _body,
            grid=(NCB,),
            in_specs=in_specs,
            out_specs=[col_spec(cout), full(cout, 1), full(cout, 1)],
            out_shape=[jax.ShapeDtypeStruct((cout, BL), jnp.float32),
                       jax.ShapeDtypeStruct((cout, 1), jnp.float32),
                       jax.ShapeDtypeStruct((cout, 1), jnp.float32)],
        )(x, *pre, w3[:, :, 0], w3[:, :, 1], w3[:, :, 2], b[:, None])

    return _call


def _make_res_combine(cin, cout):
    def _body(o2_ref, s1_ref, s2_ref, g_ref, be_ref, x_ref, wd_ref, bd_ref,
              out_ref):
        a2 = _bn_apply(o2_ref[...], s1_ref[...], s2_ref[...],
                       g_ref[...], be_ref[...])
        res = jnp.dot(wd_ref[...], x_ref[...],
                      preferred_element_type=jnp.float32) + bd_ref[...]
        out_ref[...] = jnp.maximum(a2 + res, 0.0)

    col_spec = lambda c: pl.BlockSpec((c, CB), lambda j: (0, j))
    full = lambda a, b: pl.BlockSpec((a, b), lambda j: (0, 0))

    def _call(o2, s1, s2, g, be, x, wd, bd):
        return pl.pallas_call(
            _body,
            grid=(NCB,),
            in_specs=[col_spec(cout), full(cout, 1), full(cout, 1),
                      full(cout, 1), full(cout, 1), col_spec(cin),
                      full(cout, cin), full(cout, 1)],
            out_specs=col_spec(cout),
            out_shape=jax.ShapeDtypeStruct((cout, BL), jnp.float32),
        )(o2, s1, s2, g, be, x, wd, bd)

    return _call


def _make_tblock(cin, cout, dil):
    conv1 = _make_conv_stats(cin, cout, dil, pre_bn=False)
    conv2 = _make_conv_stats(cout, cout, dil, pre_bn=True)
    comb = _make_res_combine(cin, cout)

    def _call(x, p, pre):
        o1, a1, a2 = conv1(x, p[pre + '_w1'], p[pre + '_b1'])
        o2, c1, c2 = conv2(o1, p[pre + '_w2'], p[pre + '_b2'],
                           pre=(a1, a2, p[pre + '_bn1_g'][:, None],
                                p[pre + '_bn1_b'][:, None]))
        return comb(o2, c1, c2, p[pre + '_bn2_g'][:, None],
                    p[pre + '_bn2_b'][:, None], x,
                    p[pre + '_down_w'][:, :, 0], p[pre + '_down_b'][:, None])

    return _call


_tblock0 = _make_tblock(NPG, 128, 1)
_tblock1 = _make_tblock(128, 64, 2)
_tblock2 = _make_tblock(64, 32, 4)


def _head_body(t_ref, w1_ref, b1_ref, w2_ref, b2_ref, out_ref):
    wc = jnp.dot(w1_ref[...], w2_ref[...], preferred_element_type=jnp.float32)
    bc = jnp.dot(b1_ref[...], w2_ref[...],
                 preferred_element_type=jnp.float32) + b2_ref[...]
    out_ref[...] = jnp.dot(t_ref[...], wc,
                           preferred_element_type=jnp.float32) + bc


def _head(t, w1, b1, w2, b2):
    return pl.pallas_call(
        _head_body,
        out_shape=jax.ShapeDtypeStruct((BATCH, 1), jnp.float32),
    )(t, w1, b1, w2, b2)


def kernel(x, edge_index, params):
    p = params
    src = edge_index[0]
    dst = edge_index[1]
    dst2 = dst.reshape(N_EDGES // K_EDGE, K_EDGE)

    deg_part = _deg_counts(dst)
    deg = jnp.sum(deg_part[:, :N_NODES], axis=0) + 1.0
    dis = lax.rsqrt(deg)[:, None]

    pad64 = lambda a: jnp.pad(a, ((0, 0), (0, 64)))
    g1 = _t0(x, pad64(p['gcn1_w']), dis)
    pt = _msg128(src, dst2, g1)
    g2 = _tmid(pt[0], pt[1], g1, dis, pad64(p['gcn1_b'][None, :]),
               pad64(p['bn1_g'][None, :]), pad64(p['bn1_b'][None, :]),
               jnp.pad(p['gcn2_w'], ((0, 64), (0, 0))))
    pt = _msg128(src, dst2, g2)
    g3 = _tmid(pt[0], pt[1], g2, dis, p['gcn2_b'][None, :],
               p['bn2_g'][None, :], p['bn2_b'][None, :], p['gcn3_w'])
    pt = _msg128(src, dst2, g3)
    h3 = _tlast(pt[0], pt[1], g3, dis, p['gcn3_b'][None, :],
                p['bn3_g'][None, :], p['bn3_b'][None, :])

    x0 = h3.reshape(BATCH, NPG, SEQ).transpose(1, 0, 2).reshape(NPG, BL)
    x1 = _tblock0(x0, p, 'tcn0')
    x2 = _tblock1(x1, p, 'tcn1')
    x3 = _tblock2(x2, p, 'tcn2')

    t = x3.reshape(32, BATCH, SEQ)[:, :, SEQ - 1].T
    return _head(t, p['fc1_w'], p['fc1_b'][None, :], p['fc_w'],
                 p['fc_b'][None, :])


# speedup vs baseline: 14.6683x; 1.0023x over previous
import functools

import jax
import jax.numpy as jnp
from jax import lax
from jax.experimental import pallas as pl
from jax.experimental.pallas import tpu as pltpu
from jax.experimental.pallas import tpu_sc as plsc

N_NODES = 10080
N_EDGES = 322560
BATCH = 360
NPG = 28
SEQ = 128
BL = BATCH * SEQ

NC = 2
NS = 16
NW = NC * NS
EPT = N_EDGES // NW
K_EDGE = 112
NCHUNK = EPT // K_EDGE
NPT = N_NODES // NS
NPAD = 10240
LANES = 16
EPS = 1e-5

_SC_MESH = plsc.VectorSubcoreMesh(core_axis_name="c", subcore_axis_name="s",
                                  num_cores=NC, num_subcores=NS)


@functools.partial(
    pl.kernel,
    out_type=jax.ShapeDtypeStruct((NW, NPAD), jnp.float32),
    mesh=_SC_MESH,
    scratch_types=[
        pltpu.VMEM((EPT,), jnp.int32),
        pltpu.VMEM((NPAD,), jnp.float32),
    ],
    compiler_params=pltpu.CompilerParams(needs_layout_passes=False),
)
def _deg_counts(dst_hbm, out_hbm, idx_v, hist_v):
    cid = lax.axis_index("c")
    sid = lax.axis_index("s")
    wid = cid * NS + sid

    zero16 = jnp.zeros((LANES,), jnp.float32)

    def _zero(i, c):
        hist_v[pl.ds(i * LANES, LANES)] = zero16
        return c

    lax.fori_loop(0, NPAD // LANES, _zero, 0)

    pltpu.sync_copy(dst_hbm.at[pl.ds(wid * EPT, EPT)], idx_v)

    ones16 = jnp.ones((LANES,), jnp.float32)

    def _accum(i, c):
        idx = idx_v[pl.ds(i * LANES, LANES)]
        plsc.addupdate_scatter(hist_v, [idx], ones16)
        return c

    lax.fori_loop(0, EPT // LANES, _accum, 0)

    pltpu.sync_copy(hist_v, out_hbm.at[wid])


def _make_msg_kernel(d):
    @functools.partial(
        pl.kernel,
        out_type=jax.ShapeDtypeStruct((NC, N_NODES, d), jnp.float32),
        mesh=_SC_MESH,
        scratch_types=[
            pltpu.VMEM((EPT,), jnp.int32),
            pltpu.VMEM((NCHUNK, K_EDGE), jnp.int32),
            pltpu.VMEM((K_EDGE, d), jnp.float32),
            pltpu.VMEM((K_EDGE, d), jnp.float32),
            pltpu.VMEM_SHARED((N_NODES, d), jnp.float32),
            pltpu.SemaphoreType.DMA,
            pltpu.SemaphoreType.DMA,
        ],
        compiler_params=pltpu.CompilerParams(needs_layout_passes=False,
                                             use_tc_tiling_on_sc=False),
    )
    def _msg(src_hbm, dst2_hbm, g_hbm, out_hbm, si_v, di_v, rows0_v, rows1_v,
             acc_sh, sem0, sem1):
        cid = lax.axis_index("c")
        sid = lax.axis_index("s")
        wid = cid * NS + sid
        row0 = sid * NPT

        pltpu.sync_copy(src_hbm.at[pl.ds(wid * EPT, EPT)], si_v)
        pltpu.sync_copy(dst2_hbm.at[pl.ds(wid * NCHUNK, NCHUNK)], di_v)
        pltpu.sync_copy(g_hbm.at[pl.ds(row0, NPT)], acc_sh.at[pl.ds(row0, NPT)])
        plsc.subcore_barrier()

        def _gather(j, rref, sem):
            return pltpu.make_async_copy(
                g_hbm.at[si_v.at[pl.ds(j * K_EDGE, K_EDGE)]], rref, sem)

        _gather(0, rows0_v, sem0).start()
        _gather(1, rows1_v, sem1).start()

        def _step(j, rref, sem):
            _gather(j, rref, sem).wait()
            pltpu.sync_copy(rref, acc_sh.at[di_v.at[j]], add=True)

            @pl.when(j + 2 < NCHUNK)
            def _():
                _gather(j + 2, rref, sem).start()

        def _chunk2(j2, c):
            _step(j2 * 2, rows0_v, sem0)
            _step(j2 * 2 + 1, rows1_v, sem1)
            return c

        lax.fori_loop(0, NCHUNK // 2, _chunk2, 0)
        plsc.subcore_barrier()

        pltpu.sync_copy(acc_sh.at[pl.ds(row0, NPT)],
                        out_hbm.at[cid, pl.ds(row0, NPT)])

    return _msg


_msg128 = _make_msg_kernel(128)


def _bdot(a, b):
    return jnp.dot(a.astype(jnp.bfloat16), b.astype(jnp.bfloat16),
                   preferred_element_type=jnp.float32)


def _t0_body(x_ref, w_ref, dis_ref, out_ref):
    out_ref[...] = _bdot(x_ref[...], w_ref[...]) * dis_ref[...]


def _t0(x, w, dis):
    return pl.pallas_call(
        _t0_body,
        out_shape=jax.ShapeDtypeStruct((N_NODES, w.shape[1]), jnp.float32),
    )(x, w, dis)


def _gcn_post(p0, p1, g, dis, b, gam, bet):
    s = (p0 + p1 - g) * dis + b
    m = jnp.mean(s, axis=0, keepdims=True)
    c = s - m
    v = jnp.mean(c * c, axis=0, keepdims=True)
    return jnp.maximum(gam * c * lax.rsqrt(v + EPS) + bet, 0.0)


def _tmid_body(p0_ref, p1_ref, g_ref, dis_ref, b_ref, gam_ref, bet_ref,
               w_ref, out_ref):
    h = _gcn_post(p0_ref[...], p1_ref[...], g_ref[...], dis_ref[...],
                  b_ref[...], gam_ref[...], bet_ref[...])
    out_ref[...] = _bdot(h, w_ref[...]) * dis_ref[...]


def _tmid(p0, p1, g, dis, b, gam, bet, w):
    return pl.pallas_call(
        _tmid_body,
        out_shape=jax.ShapeDtypeStruct((N_NODES, w.shape[1]), jnp.float32),
    )(p0, p1, g, dis, b, gam, bet, w)


def _tlast_body(p0_ref, p1_ref, g_ref, dis_ref, b_ref, gam_ref, bet_ref,
                out_ref):
    out_ref[...] = _gcn_post(p0_ref[...], p1_ref[...], g_ref[...], dis_ref[...],
                             b_ref[...], gam_ref[...], bet_ref[...])


def _tlast(p0, p1, g, dis, b, gam, bet):
    return pl.pallas_call(
        _tlast_body,
        out_shape=jax.ShapeDtypeStruct((N_NODES, 128), jnp.float32),
    )(p0, p1, g, dis, b, gam, bet)


CB = 5760
NCB = BL // CB


def _shift_cols(x, s):
    if s == 0:
        return x
    w = x.shape[1]
    z = jnp.zeros((x.shape[0], s), jnp.float32)
    xs = jnp.concatenate([z, x[:, :w - s]], axis=1)
    col = lax.broadcasted_iota(jnp.int32, (1, w), 1)
    return xs * (col % SEQ >= s).astype(jnp.float32)


def _conv_cf(x, wk, b, dil):
    o = b
    for k in range(3):
        o = o + jnp.dot(wk[k], _shift_cols(x, (2 - k) * dil),
                        preferred_element_type=jnp.float32)
    return o


def _bn_apply(x, s1, s2, gam, bet):
    m = s1 * (1.0 / BL)
    v = s2 * (1.0 / BL) - m * m
    return jnp.maximum(gam * (x - m) * lax.rsqrt(v + EPS) + bet, 0.0)


def _make_conv_stats(cin, cout, dil, pre_bn):
    def _body(*refs):
        if pre_bn:
            (x_ref, ps1_ref, ps2_ref, pg_ref, pb_ref,
             w0_ref, w1_ref, w2_ref, b_ref, o_ref, s1_ref, s2_ref) = refs
            x = _bn_apply(x_ref[...], ps1_ref[...], ps2_ref[...],
                          pg_ref[...], pb_ref[...])
        else:
            x_ref, w0_ref, w1_ref, w2_ref, b_ref, o_ref, s1_ref, s2_ref = refs
            x = x_ref[...]
        o = _conv_cf(x, [w0_ref[...], w1_ref[...], w2_ref[...]],
                     b_ref[...], dil)
        o_ref[...] = o

        @pl.when(pl.program_id(0) == 0)
        def _():
            s1_ref[...] = jnp.zeros_like(s1_ref)
            s2_ref[...] = jnp.zeros_like(s2_ref)

        s1_ref[...] += jnp.sum(o, axis=1, keepdims=True)
        s2_ref[...] += jnp.sum(o * o, axis=1, keepdims=True)

    col_spec = lambda c: pl.BlockSpec((c, CB), lambda j: (0, j))
    full = lambda a, b: pl.BlockSpec((a, b), lambda j: (0, 0))
    in_specs = [col_spec(cin)]
    if pre_bn:
        in_specs += [full(cin, 1)] * 4
    in_specs += [full(cout, cin)] * 3 + [full(cout, 1)]

    def _call(x, w3, b, pre=()):
        return pl.pallas_call(
            _body,
            grid=(NCB,),
            in_specs=in_specs,
            out_specs=[col_spec(cout), full(cout, 1), full(cout, 1)],
            out_shape=[jax.ShapeDtypeStruct((cout, BL), jnp.float32),
                       jax.ShapeDtypeStruct((cout, 1), jnp.float32),
                       jax.ShapeDtypeStruct((cout, 1), jnp.float32)],
        )(x, *pre, w3[:, :, 0], w3[:, :, 1], w3[:, :, 2], b[:, None])

    return _call


def _make_res_combine(cin, cout):
    def _body(o2_ref, s1_ref, s2_ref, g_ref, be_ref, x_ref, wd_ref, bd_ref,
              out_ref):
        a2 = _bn_apply(o2_ref[...], s1_ref[...], s2_ref[...],
                       g_ref[...], be_ref[...])
        res = jnp.dot(wd_ref[...], x_ref[...],
                      preferred_element_type=jnp.float32) + bd_ref[...]
        out_ref[...] = jnp.maximum(a2 + res, 0.0)

    col_spec = lambda c: pl.BlockSpec((c, CB), lambda j: (0, j))
    full = lambda a, b: pl.BlockSpec((a, b), lambda j: (0, 0))

    def _call(o2, s1, s2, g, be, x, wd, bd):
        return pl.pallas_call(
            _body,
            grid=(NCB,),
            in_specs=[col_spec(cout), full(cout, 1), full(cout, 1),
                      full(cout, 1), full(cout, 1), col_spec(cin),
                      full(cout, cin), full(cout, 1)],
            out_specs=col_spec(cout),
            out_shape=jax.ShapeDtypeStruct((cout, BL), jnp.float32),
        )(o2, s1, s2, g, be, x, wd, bd)

    return _call


def _make_tblock(cin, cout, dil):
    conv1 = _make_conv_stats(cin, cout, dil, pre_bn=False)
    conv2 = _make_conv_stats(cout, cout, dil, pre_bn=True)
    comb = _make_res_combine(cin, cout)

    def _call(x, p, pre):
        o1, a1, a2 = conv1(x, p[pre + '_w1'], p[pre + '_b1'])
        o2, c1, c2 = conv2(o1, p[pre + '_w2'], p[pre + '_b2'],
                           pre=(a1, a2, p[pre + '_bn1_g'][:, None],
                                p[pre + '_bn1_b'][:, None]))
        return comb(o2, c1, c2, p[pre + '_bn2_g'][:, None],
                    p[pre + '_bn2_b'][:, None], x,
                    p[pre + '_down_w'][:, :, 0], p[pre + '_down_b'][:, None])

    return _call


_tblock0 = _make_tblock(NPG, 128, 1)
_tblock1 = _make_tblock(128, 64, 2)
_tblock2 = _make_tblock(64, 32, 4)


def _head_body(t_ref, w1_ref, b1_ref, w2_ref, b2_ref, out_ref):
    wc = jnp.dot(w1_ref[...], w2_ref[...], preferred_element_type=jnp.float32)
    bc = jnp.dot(b1_ref[...], w2_ref[...],
                 preferred_element_type=jnp.float32) + b2_ref[...]
    out_ref[...] = jnp.dot(t_ref[...], wc,
                           preferred_element_type=jnp.float32) + bc


def _head(t, w1, b1, w2, b2):
    return pl.pallas_call(
        _head_body,
        out_shape=jax.ShapeDtypeStruct((BATCH, 1), jnp.float32),
    )(t, w1, b1, w2, b2)


def kernel(x, edge_index, params):
    p = params
    src = edge_index[0]
    dst = edge_index[1]
    dst2 = dst.reshape(N_EDGES // K_EDGE, K_EDGE)

    deg_part = _deg_counts(dst)
    deg = jnp.sum(deg_part[:, :N_NODES], axis=0) + 1.0
    dis = lax.rsqrt(deg)[:, None]

    pad64 = lambda a: jnp.pad(a, ((0, 0), (0, 64)))
    g1 = _t0(x, pad64(p['gcn1_w']), dis)
    pt = _msg128(src, dst2, g1)
    g2 = _tmid(pt[0], pt[1], g1, dis, pad64(p['gcn1_b'][None, :]),
               pad64(p['bn1_g'][None, :]), pad64(p['bn1_b'][None, :]),
               jnp.pad(p['gcn2_w'], ((0, 64), (0, 0))))
    pt = _msg128(src, dst2, g2)
    g3 = _tmid(pt[0], pt[1], g2, dis, p['gcn2_b'][None, :],
               p['bn2_g'][None, :], p['bn2_b'][None, :], p['gcn3_w'])
    pt = _msg128(src, dst2, g3)
    h3 = _tlast(pt[0], pt[1], g3, dis, p['gcn3_b'][None, :],
                p['bn3_g'][None, :], p['bn3_b'][None, :])

    x0 = h3.reshape(BATCH, NPG, SEQ).transpose(1, 0, 2).reshape(NPG, BL)
    x1 = _tblock0(x0, p, 'tcn0')
    x2 = _tblock1(x1, p, 'tcn1')
    x3 = _tblock2(x2, p, 'tcn2')

    t = x3.reshape(32, BATCH, SEQ)[:, :, SEQ - 1].T
    return _head(t, p['fc1_w'], p['fc1_b'][None, :], p['fc_w'],
                 p['fc_b'][None, :])
